# speedup vs baseline: 2.9664x; 2.9664x over previous
import functools

import jax
import jax.numpy as jnp
from jax import lax
from jax.experimental import pallas as pl
from jax.experimental.pallas import tpu as pltpu
from jax.experimental.pallas import tpu_sc as plsc

NUM_EMBEDDINGS = 100000
EMBEDDING_DIM = 128

_info = plsc.get_sparse_core_info()
_NC, _NS = _info.num_cores, _info.num_subcores
_NW = _NC * _NS

_B = 4096 * 50
_B_PER_W = _B // _NW
_CHUNK = 128
_NCHUNKS = _B_PER_W // _CHUNK


def _gather_body(idx_hbm, table_hbm, out_hbm, idx_v, rows_v, sem):
    wid = lax.axis_index("s") * _NC + lax.axis_index("c")
    base = wid * _B_PER_W
    pltpu.sync_copy(idx_hbm.at[pl.ds(base, _B_PER_W)], idx_v)

    def step(c, carry):
        cp = pltpu.async_copy(
            table_hbm.at[idx_v.at[pl.ds(c * _CHUNK, _CHUNK)]], rows_v, sem
        )
        cp.wait()
        pltpu.sync_copy(rows_v, out_hbm.at[pl.ds(base + c * _CHUNK, _CHUNK)])
        return carry

    lax.fori_loop(0, _NCHUNKS, step, 0)


@jax.jit
def _lookup(flat_ids, table):
    mesh = plsc.VectorSubcoreMesh(core_axis_name="c", subcore_axis_name="s")
    return pl.kernel(
        _gather_body,
        mesh=mesh,
        out_type=jax.ShapeDtypeStruct((_B, EMBEDDING_DIM), jnp.float32),
        scratch_types=[
            pltpu.VMEM((_B_PER_W,), jnp.int32),
            pltpu.VMEM((_CHUNK, EMBEDDING_DIM), jnp.float32),
            pltpu.SemaphoreType.DMA,
        ],
    )(flat_ids, table)


def kernel(token_ids, embedding_map):
    shp = token_ids.shape
    flat = token_ids.reshape(-1).astype(jnp.int32)
    out = _lookup(flat, embedding_map)
    return out.reshape(*shp, EMBEDDING_DIM)


# speedup vs baseline: 3.3303x; 1.1227x over previous
import functools

import jax
import jax.numpy as jnp
from jax import lax
from jax.experimental import pallas as pl
from jax.experimental.pallas import tpu as pltpu
from jax.experimental.pallas import tpu_sc as plsc

NUM_EMBEDDINGS = 100000
EMBEDDING_DIM = 128

_info = plsc.get_sparse_core_info()
_NC, _NS = _info.num_cores, _info.num_subcores
_NW = _NC * _NS

_B = 4096 * 50
_B_PER_W = _B // _NW
_CHUNK = 128
_NCHUNKS = _B_PER_W // _CHUNK


def _gather_body(idx_hbm, table_hbm, out_hbm, idx_v, rows0, rows1, sem0, sem1):
    wid = lax.axis_index("s") * _NC + lax.axis_index("c")
    base = wid * _B_PER_W
    pltpu.sync_copy(idx_hbm.at[pl.ds(base, _B_PER_W)], idx_v)

    bufs = (rows0, rows1)
    sems = (sem0, sem1)

    def start_gather(c, buf, sem):
        pltpu.async_copy(table_hbm.at[idx_v.at[pl.ds(c * _CHUNK, _CHUNK)]], buf, sem)

    start_gather(0, bufs[0], sems[0])

    def outer(c0):
        for b in range(2):
            c = c0 + b

            @pl.when(c + 1 < _NCHUNKS)
            def _():
                start_gather(c + 1, bufs[1 - b], sems[1 - b])

            pltpu.make_async_copy(
                table_hbm.at[idx_v.at[pl.ds(c * _CHUNK, _CHUNK)]], bufs[b], sems[b]
            ).wait()
            pltpu.sync_copy(bufs[b], out_hbm.at[pl.ds(base + c * _CHUNK, _CHUNK)])

    pl.loop(0, _NCHUNKS, step=2)(outer)


@jax.jit
def _lookup(flat_ids, table):
    mesh = plsc.VectorSubcoreMesh(core_axis_name="c", subcore_axis_name="s")
    return pl.kernel(
        _gather_body,
        mesh=mesh,
        out_type=jax.ShapeDtypeStruct((_B, EMBEDDING_DIM), jnp.float32),
        scratch_types=[
            pltpu.VMEM((_B_PER_W,), jnp.int32),
            pltpu.VMEM((_CHUNK, EMBEDDING_DIM), jnp.float32),
            pltpu.VMEM((_CHUNK, EMBEDDING_DIM), jnp.float32),
            pltpu.SemaphoreType.DMA,
            pltpu.SemaphoreType.DMA,
        ],
    )(flat_ids, table)


def kernel(token_ids, embedding_map):
    shp = token_ids.shape
    flat = token_ids.reshape(-1).astype(jnp.int32)
    out = _lookup(flat, embedding_map)
    return out.reshape(*shp, EMBEDDING_DIM)


# speedup vs baseline: 3.3387x; 1.0025x over previous
import functools

import jax
import jax.numpy as jnp
from jax import lax
from jax.experimental import pallas as pl
from jax.experimental.pallas import tpu as pltpu
from jax.experimental.pallas import tpu_sc as plsc

NUM_EMBEDDINGS = 100000
EMBEDDING_DIM = 128

_info = plsc.get_sparse_core_info()
_NC, _NS = _info.num_cores, _info.num_subcores
_NW = _NC * _NS

_B = 4096 * 50
_B_PER_W = _B // _NW
_CHUNK = 128
_NCHUNKS = _B_PER_W // _CHUNK


_NBUF = 5


def _gather_body(idx_hbm, table_hbm, out_hbm, idx_v, *rest):
    bufs = rest[:_NBUF]
    sems = rest[_NBUF:]
    wid = lax.axis_index("s") * _NC + lax.axis_index("c")
    base = wid * _B_PER_W
    pltpu.sync_copy(idx_hbm.at[pl.ds(base, _B_PER_W)], idx_v)

    def start_gather(c, buf, sem):
        pltpu.async_copy(table_hbm.at[idx_v.at[pl.ds(c * _CHUNK, _CHUNK)]], buf, sem)

    for b in range(_NBUF):
        start_gather(b, bufs[b], sems[b])

    def outer(c0):
        for b in range(_NBUF):
            c = c0 + b
            pltpu.make_async_copy(
                table_hbm.at[idx_v.at[pl.ds(c * _CHUNK, _CHUNK)]], bufs[b], sems[b]
            ).wait()
            pltpu.sync_copy(bufs[b], out_hbm.at[pl.ds(base + c * _CHUNK, _CHUNK)])

            @pl.when(c + _NBUF < _NCHUNKS)
            def _():
                start_gather(c + _NBUF, bufs[b], sems[b])

    pl.loop(0, _NCHUNKS, step=_NBUF)(outer)


@jax.jit
def _lookup(flat_ids, table):
    mesh = plsc.VectorSubcoreMesh(core_axis_name="c", subcore_axis_name="s")
    return pl.kernel(
        _gather_body,
        mesh=mesh,
        out_type=jax.ShapeDtypeStruct((_B, EMBEDDING_DIM), jnp.float32),
        scratch_types=(
            [pltpu.VMEM((_B_PER_W,), jnp.int32)]
            + [pltpu.VMEM((_CHUNK, EMBEDDING_DIM), jnp.float32)] * _NBUF
            + [pltpu.SemaphoreType.DMA] * _NBUF
        ),
    )(flat_ids, table)


def kernel(token_ids, embedding_map):
    shp = token_ids.shape
    flat = token_ids.reshape(-1).astype(jnp.int32)
    out = _lookup(flat, embedding_map)
    return out.reshape(*shp, EMBEDDING_DIM)


# speedup vs baseline: 5.9225x; 1.7739x over previous
import functools

import jax
import jax.numpy as jnp
from jax import lax
from jax.experimental import pallas as pl
from jax.experimental.pallas import tpu as pltpu
from jax.experimental.pallas import tpu_sc as plsc

NUM_EMBEDDINGS = 100000
EMBEDDING_DIM = 128

_info = plsc.get_sparse_core_info()
_NC, _NS = _info.num_cores, _info.num_subcores
_NW = _NC * _NS

_NSEQ = 4096
_SEQLEN = 50
_SEQ_PER_W = _NSEQ // _NW
_SBLK = 4
_NBLK = _SEQ_PER_W // _SBLK
_NBUF = 2


def _gather_body(idx_hbm, table_hbm, out_hbm, idx_v, rows0, rows1, sem0, sem1):
    bufs = (rows0, rows1)
    sems = (sem0, sem1)
    wid = lax.axis_index("s") * _NC + lax.axis_index("c")
    sbase = wid * _SEQ_PER_W
    pltpu.sync_copy(idx_hbm.at[pl.ds(sbase, _SEQ_PER_W)], idx_v)

    def start_gathers(c, buf, sem):
        for j in range(_SBLK):
            pltpu.async_copy(table_hbm.at[idx_v.at[c * _SBLK + j]], buf.at[j], sem)

    def wait_gathers(c, buf, sem):
        for j in range(_SBLK):
            pltpu.make_async_copy(
                table_hbm.at[idx_v.at[c * _SBLK + j]], buf.at[j], sem
            ).wait()

    for b in range(_NBUF):
        start_gathers(b, bufs[b], sems[b])

    def outer(c0):
        for b in range(_NBUF):
            c = c0 + b
            wait_gathers(c, bufs[b], sems[b])
            pltpu.sync_copy(bufs[b], out_hbm.at[pl.ds(sbase + c * _SBLK, _SBLK)])

            @pl.when(c + _NBUF < _NBLK)
            def _():
                start_gathers(c + _NBUF, bufs[b], sems[b])

    pl.loop(0, _NBLK, step=_NBUF)(outer)


@jax.jit
def _lookup(ids2d, table):
    mesh = plsc.VectorSubcoreMesh(core_axis_name="c", subcore_axis_name="s")
    return pl.kernel(
        _gather_body,
        mesh=mesh,
        out_type=jax.ShapeDtypeStruct((_NSEQ, _SEQLEN, EMBEDDING_DIM), jnp.float32),
        compiler_params=pltpu.CompilerParams(use_tc_tiling_on_sc=True),
        scratch_types=(
            [pltpu.VMEM((_SEQ_PER_W, _SEQLEN), jnp.int32)]
            + [pltpu.VMEM((_SBLK, _SEQLEN, EMBEDDING_DIM), jnp.float32)] * _NBUF
            + [pltpu.SemaphoreType.DMA] * _NBUF
        ),
    )(ids2d, table)


def kernel(token_ids, embedding_map):
    ids2d = token_ids.astype(jnp.int32)
    return _lookup(ids2d, embedding_map)


# speedup vs baseline: 10.3568x; 1.7487x over previous
import jax
import jax.numpy as jnp
from jax import lax
from jax.experimental import pallas as pl
from jax.experimental.pallas import tpu as pltpu
from jax.experimental.pallas import tpu_sc as plsc

NUM_EMBEDDINGS = 100000
EMBEDDING_DIM = 128

_info = plsc.get_sparse_core_info()
_NC, _NS = _info.num_cores, _info.num_subcores
_NW = _NC * _NS

_B = 4096 * 50
_B_PER_W = _B // _NW
_CHUNK = 128
_NCHUNKS = _B_PER_W // _CHUNK
_NBUF = 2


def _gather_body(idx_hbm, table_hbm, out_hbm, idx_v, *rest):
    bufs = rest[:_NBUF]
    sems = rest[_NBUF:]
    wid = lax.axis_index("s") * _NC + lax.axis_index("c")
    base = wid * _B_PER_W
    pltpu.sync_copy(idx_hbm.at[pl.ds(base, _B_PER_W)], idx_v)

    def start_gather(c, buf, sem):
        pltpu.async_copy(table_hbm.at[idx_v.at[pl.ds(c * _CHUNK, _CHUNK)]], buf, sem)

    for b in range(_NBUF):
        start_gather(b, bufs[b], sems[b])

    def outer(c0):
        for b in range(_NBUF):
            c = c0 + b
            pltpu.make_async_copy(
                table_hbm.at[idx_v.at[pl.ds(c * _CHUNK, _CHUNK)]], bufs[b], sems[b]
            ).wait()
            pltpu.sync_copy(bufs[b], out_hbm.at[pl.ds(base + c * _CHUNK, _CHUNK)])

            @pl.when(c + _NBUF < _NCHUNKS)
            def _():
                start_gather(c + _NBUF, bufs[b], sems[b])

    pl.loop(0, _NCHUNKS, step=_NBUF)(outer)


@jax.jit
def _lookup(flat_ids, table):
    mesh = plsc.VectorSubcoreMesh(core_axis_name="c", subcore_axis_name="s")
    return pl.kernel(
        _gather_body,
        mesh=mesh,
        out_type=jax.ShapeDtypeStruct((_B, EMBEDDING_DIM), jnp.float32),
        scratch_types=(
            [pltpu.VMEM((_B_PER_W,), jnp.int32)]
            + [pltpu.VMEM((_CHUNK, EMBEDDING_DIM), jnp.float32)] * _NBUF
            + [pltpu.SemaphoreType.DMA] * _NBUF
        ),
    )(flat_ids, table)


def kernel(token_ids, embedding_map):
    nseq, seqlen = token_ids.shape
    flat_t = token_ids.astype(jnp.int32).T.reshape(-1)
    rows = _lookup(flat_t, embedding_map)
    return rows.reshape(seqlen, nseq, EMBEDDING_DIM).transpose(1, 0, 2)


# speedup vs baseline: 10.4601x; 1.0100x over previous
import jax
import jax.numpy as jnp
from jax import lax
from jax.experimental import pallas as pl
from jax.experimental.pallas import tpu as pltpu
from jax.experimental.pallas import tpu_sc as plsc

NUM_EMBEDDINGS = 100000
EMBEDDING_DIM = 128

_info = plsc.get_sparse_core_info()
_NC, _NS = _info.num_cores, _info.num_subcores
_NW = _NC * _NS

_NSEQ = 4096
_SEQLEN = 50
_SEQ_PER_W = _NSEQ // _NW
_NBUF = 2


def _gather_body(idx_hbm, table_hbm, out_hbm, idx_v, *rest):
    bufs = rest[:_NBUF]
    sems = rest[_NBUF:]
    wid = lax.axis_index("s") * _NC + lax.axis_index("c")
    sbase = wid * _SEQ_PER_W
    pltpu.sync_copy(idx_hbm.at[:, pl.ds(sbase, _SEQ_PER_W)], idx_v)

    def start_gather(j, buf, sem):
        pltpu.async_copy(table_hbm.at[idx_v.at[j]], buf, sem)

    for b in range(_NBUF):
        start_gather(b, bufs[b], sems[b])

    def outer(j0):
        for b in range(_NBUF):
            j = j0 + b
            pltpu.make_async_copy(
                table_hbm.at[idx_v.at[j]], bufs[b], sems[b]
            ).wait()
            pltpu.sync_copy(bufs[b], out_hbm.at[j, pl.ds(sbase, _SEQ_PER_W)])

            @pl.when(j + _NBUF < _SEQLEN)
            def _():
                start_gather(j + _NBUF, bufs[b], sems[b])

    pl.loop(0, _SEQLEN, step=_NBUF)(outer)


@jax.jit
def _lookup(ids_t, table):
    mesh = plsc.VectorSubcoreMesh(core_axis_name="c", subcore_axis_name="s")
    return pl.kernel(
        _gather_body,
        mesh=mesh,
        out_type=jax.ShapeDtypeStruct((_SEQLEN, _NSEQ, EMBEDDING_DIM), jnp.float32),
        scratch_types=(
            [pltpu.VMEM((_SEQLEN, _SEQ_PER_W), jnp.int32)]
            + [pltpu.VMEM((_SEQ_PER_W, EMBEDDING_DIM), jnp.float32)] * _NBUF
            + [pltpu.SemaphoreType.DMA] * _NBUF
        ),
    )(ids_t, table)


def kernel(token_ids, embedding_map):
    ids_t = token_ids.astype(jnp.int32).T
    rows = _lookup(ids_t, embedding_map)
    return rows.transpose(1, 0, 2)


# speedup vs baseline: 10.7230x; 1.0251x over previous
import jax
import jax.numpy as jnp
from jax import lax
from jax.experimental import pallas as pl
from jax.experimental.pallas import tpu as pltpu
from jax.experimental.pallas import tpu_sc as plsc

NUM_EMBEDDINGS = 100000
EMBEDDING_DIM = 128

_info = plsc.get_sparse_core_info()
_NC, _NS = _info.num_cores, _info.num_subcores
_NW = _NC * _NS

_NSEQ = 4096
_SEQLEN = 50
_SEQ_PER_W = _NSEQ // _NW
_NBUF = 5


def _gather_body(idx_hbm, table_hbm, out_hbm, idx_v, *rest):
    bufs = rest[:_NBUF]
    sems = rest[_NBUF:]
    wid = lax.axis_index("s") * _NC + lax.axis_index("c")
    sbase = wid * _SEQ_PER_W
    pltpu.sync_copy(idx_hbm.at[:, pl.ds(sbase, _SEQ_PER_W)], idx_v)

    def start_gather(j, buf, sem):
        pltpu.async_copy(table_hbm.at[idx_v.at[j]], buf, sem)

    for b in range(_NBUF):
        start_gather(b, bufs[b], sems[b])

    def outer(j0):
        for b in range(_NBUF):
            j = j0 + b
            pltpu.make_async_copy(
                table_hbm.at[idx_v.at[j]], bufs[b], sems[b]
            ).wait()
            pltpu.sync_copy(bufs[b], out_hbm.at[j, pl.ds(sbase, _SEQ_PER_W)])

            @pl.when(j + _NBUF < _SEQLEN)
            def _():
                start_gather(j + _NBUF, bufs[b], sems[b])

    pl.loop(0, _SEQLEN, step=_NBUF)(outer)


@jax.jit
def _lookup(ids_t, table):
    mesh = plsc.VectorSubcoreMesh(core_axis_name="c", subcore_axis_name="s")
    return pl.kernel(
        _gather_body,
        mesh=mesh,
        out_type=jax.ShapeDtypeStruct((_SEQLEN, _NSEQ, EMBEDDING_DIM), jnp.float32),
        scratch_types=(
            [pltpu.VMEM((_SEQLEN, _SEQ_PER_W), jnp.int32)]
            + [pltpu.VMEM((_SEQ_PER_W, EMBEDDING_DIM), jnp.float32)] * _NBUF
            + [pltpu.SemaphoreType.DMA] * _NBUF
        ),
    )(ids_t, table)


def kernel(token_ids, embedding_map):
    ids_t = token_ids.astype(jnp.int32).T
    rows = _lookup(ids_t, embedding_map)
    return rows.transpose(1, 0, 2)


# speedup vs baseline: 10.7754x; 1.0049x over previous
import jax
import jax.numpy as jnp
from jax import lax
from jax.experimental import pallas as pl
from jax.experimental.pallas import tpu as pltpu
from jax.experimental.pallas import tpu_sc as plsc

NUM_EMBEDDINGS = 100000
EMBEDDING_DIM = 128

_info = plsc.get_sparse_core_info()
_NC, _NS = _info.num_cores, _info.num_subcores
_NW = _NC * _NS

_NSEQ = 4096
_SEQLEN = 50
_SEQ_PER_W = _NSEQ // _NW
_HALF = _SEQ_PER_W // 2
_NUNITS = _SEQLEN * 2
_NBUF = 10


def _gather_body(idx_hbm, table_hbm, out_hbm, idx_v, *rest):
    bufs = rest[:_NBUF]
    sems = rest[_NBUF:]
    wid = lax.axis_index("s") * _NC + lax.axis_index("c")
    sbase = wid * _SEQ_PER_W
    pltpu.sync_copy(idx_hbm.at[:, pl.ds(sbase, _SEQ_PER_W)], idx_v)

    def start_gather(u, buf, sem):
        j, h = u // 2, u % 2
        pltpu.async_copy(
            table_hbm.at[idx_v.at[j, pl.ds(h * _HALF, _HALF)]], buf, sem
        )

    def wait_gather(u, buf, sem):
        j, h = u // 2, u % 2
        pltpu.make_async_copy(
            table_hbm.at[idx_v.at[j, pl.ds(h * _HALF, _HALF)]], buf, sem
        ).wait()

    for b in range(_NBUF):
        start_gather(b, bufs[b], sems[b])

    def outer(u0):
        for b in range(_NBUF):
            u = u0 + b
            wait_gather(u, bufs[b], sems[b])
            j, h = u // 2, u % 2
            pltpu.sync_copy(
                bufs[b], out_hbm.at[j, pl.ds(sbase + h * _HALF, _HALF)]
            )

            @pl.when(u + _NBUF < _NUNITS)
            def _():
                start_gather(u + _NBUF, bufs[b], sems[b])

    pl.loop(0, _NUNITS, step=_NBUF)(outer)


@jax.jit
def _lookup(ids_t, table):
    mesh = plsc.VectorSubcoreMesh(core_axis_name="c", subcore_axis_name="s")
    return pl.kernel(
        _gather_body,
        mesh=mesh,
        out_type=jax.ShapeDtypeStruct((_SEQLEN, _NSEQ, EMBEDDING_DIM), jnp.float32),
        scratch_types=(
            [pltpu.VMEM((_SEQLEN, _SEQ_PER_W), jnp.int32)]
            + [pltpu.VMEM((_HALF, EMBEDDING_DIM), jnp.float32)] * _NBUF
            + [pltpu.SemaphoreType.DMA] * _NBUF
        ),
    )(ids_t, table)


def kernel(token_ids, embedding_map):
    ids_t = token_ids.astype(jnp.int32).T
    rows = _lookup(ids_t, embedding_map)
    return rows.transpose(1, 0, 2)
